# Initial kernel scaffold; baseline (speedup 1.0000x reference)
import jax
import jax.numpy as jnp
from jax.experimental import pallas as pl


def kernel(x, edge_index, batch, B0_W1, B0_b1, B0_W2, B0_b2, B0_Wl, B0_bl, B1_W1, B1_b1, B1_W2, B1_b2, B1_Wl, B1_bl, B2_W1, B2_b1, B2_W2, B2_b2, B2_Wl, B2_bl, lin1_W, lin1_b, lin2_W, lin2_b):
    raise NotImplementedError("write your pallas kernel here")


# speedup vs baseline: 11.3663x; 11.3663x over previous
import functools

import jax
import jax.numpy as jnp
from jax import lax
from jax.experimental import pallas as pl
from jax.experimental.pallas import tpu as pltpu
from jax.experimental.pallas import tpu_sc as plsc

SDS = jax.ShapeDtypeStruct
F32 = jnp.float32
I32 = jnp.int32

NSC = 2
NSS = 16
C = 128
RB = 1024
NEG = -3.0e38

_mesh = functools.partial(
    plsc.VectorSubcoreMesh,
    core_axis_name="c", subcore_axis_name="s", num_cores=NSC, num_subcores=NSS)


def _sc_degree(dst_r, np_rows):
    chunks = dst_r.shape[0]
    pdc = chunks // (NSC * NSS)
    rps = np_rows // NSS

    @functools.partial(
        pl.kernel,
        out_type=SDS((NSC, np_rows), F32),
        mesh=_mesh(),
        scratch_types=[
            pltpu.VMEM((pdc, C), I32),
            pltpu.VMEM((C,), F32),
            pltpu.VMEM((rps,), F32),
            pltpu.VMEM_SHARED((np_rows,), F32),
        ],
    )
    def k(dst_hbm, out_hbm, dst_v, ones_v, init_v, acc):
        cid = lax.axis_index("c")
        sid = lax.axis_index("s")
        wid = sid * NSC + cid
        pltpu.sync_copy(dst_hbm.at[pl.ds(wid * pdc, pdc)], dst_v)
        initval = jnp.where(cid == 0, 1.0, 0.0).astype(F32)
        for i in range(C // 16):
            ones_v[pl.ds(i * 16, 16)] = jnp.ones((16,), F32)

        def fill(i, carry):
            init_v[pl.ds(i * 16, 16)] = jnp.full((16,), initval, F32)
            return carry
        lax.fori_loop(0, rps // 16, fill, 0)
        pltpu.sync_copy(init_v, acc.at[pl.ds(sid * rps, rps)])
        plsc.subcore_barrier()

        def body(j, carry):
            pltpu.sync_copy(ones_v, acc.at[dst_v.at[j]], add=True)
            return carry
        lax.fori_loop(0, pdc, body, 0)
        plsc.subcore_barrier()
        pltpu.sync_copy(acc.at[pl.ds(sid * rps, rps)],
                        out_hbm.at[cid, pl.ds(sid * rps, rps)])

    return k(dst_r)


def _sc_conv(xw2, src_r, dst_r):
    chunks = src_r.shape[0]
    np_rows, hh = xw2.shape[1], xw2.shape[2]
    psc = chunks // NSS
    hc = psc // 2
    rps = np_rows // NSS

    @functools.partial(
        pl.kernel,
        out_type=SDS((NSC, np_rows, hh), F32),
        mesh=_mesh(),
        scratch_types=[
            pltpu.VMEM((hc, C), I32),
            pltpu.VMEM((hc, C), I32),
            pltpu.VMEM((C, hh), F32),
            pltpu.VMEM_SHARED((np_rows, hh), F32),
            pltpu.SemaphoreType.DMA,
        ],
    )
    def k(xw_hbm, src_hbm, dst_hbm, out_hbm, src_v, dst_v, rows_v, acc, sem):
        cid = lax.axis_index("c")
        sid = lax.axis_index("s")
        pltpu.sync_copy(xw_hbm.at[cid, pl.ds(sid * rps, rps)],
                        acc.at[pl.ds(sid * rps, rps)])
        plsc.subcore_barrier()

        def body(j, carry):
            pltpu.async_copy(xw_hbm.at[cid].at[src_v.at[j]], rows_v, sem).wait()
            pltpu.sync_copy(rows_v, acc.at[dst_v.at[j]], add=True)
            return carry

        for half in range(2):
            base = sid * psc + half * hc
            pltpu.sync_copy(src_hbm.at[pl.ds(base, hc)], src_v)
            pltpu.sync_copy(dst_hbm.at[pl.ds(base, hc)], dst_v)
            lax.fori_loop(0, hc, body, 0)
        plsc.subcore_barrier()
        pltpu.sync_copy(acc.at[pl.ds(sid * rps, rps)],
                        out_hbm.at[cid, pl.ds(sid * rps, rps)])

    return k(xw2, src_r, dst_r)


def _sc_pool_gather(h, idx_flat):
    b, hfull = idx_flat.shape[0], h.shape[1]
    bpw = b // (NSC * NSS)

    @functools.partial(
        pl.kernel,
        out_type=SDS((b, hfull), F32),
        mesh=_mesh(),
        scratch_types=[
            pltpu.VMEM((bpw,), I32),
            pltpu.VMEM((bpw, hfull), F32),
            pltpu.SemaphoreType.DMA,
        ],
    )
    def k(h_hbm, idx_hbm, out_hbm, idx_v, rows_v, sem):
        cid = lax.axis_index("c")
        sid = lax.axis_index("s")
        wid = sid * NSC + cid
        base = wid * bpw
        pltpu.sync_copy(idx_hbm.at[pl.ds(base, bpw)], idx_v)
        pltpu.async_copy(h_hbm.at[idx_v], rows_v, sem).wait()
        pltpu.sync_copy(rows_v, out_hbm.at[pl.ds(base, bpw)])

    return k(h, idx_flat)


def _relu(v):
    return jnp.maximum(v, 0.0)


def _mm_scale(h, deg, w):
    np_rows, fin = h.shape
    hout = w.shape[1]
    hh = hout // 2
    grid = np_rows // RB

    def body(h_ref, d_ref, w_ref, u_ref):
        dinv = lax.rsqrt(d_ref[...])
        r = jnp.dot(h_ref[...] * dinv, w_ref[...],
                    preferred_element_type=F32)
        u_ref[0] = r[:, :hh]
        u_ref[1] = r[:, hh:]

    return pl.pallas_call(
        body,
        grid=(grid,),
        in_specs=[
            pl.BlockSpec((RB, fin), lambda i: (i, 0)),
            pl.BlockSpec((RB, 1), lambda i: (i, 0)),
            pl.BlockSpec((fin, hout), lambda i: (0, 0)),
        ],
        out_specs=pl.BlockSpec((2, RB, hh), lambda i: (0, i, 0)),
        out_shape=SDS((2, np_rows, hh), F32),
    )(h, deg, w)


def _act_mm(s2, deg, b, w):
    np_rows, hh = s2.shape[1], s2.shape[2]
    hfull = 2 * hh
    grid = np_rows // RB

    def body(s_ref, d_ref, b_ref, w_ref, x_ref, u_ref):
        dinv = lax.rsqrt(d_ref[...])
        x0 = _relu(s_ref[0] * dinv + b_ref[0:1, :hh])
        x1 = _relu(s_ref[1] * dinv + b_ref[0:1, hh:])
        xa = jnp.concatenate([x0, x1], axis=1)
        x_ref[...] = xa
        r = jnp.dot(xa * dinv, w_ref[...], preferred_element_type=F32)
        u_ref[0] = r[:, :hh]
        u_ref[1] = r[:, hh:]

    return pl.pallas_call(
        body,
        grid=(grid,),
        in_specs=[
            pl.BlockSpec((2, RB, hh), lambda i: (0, i, 0)),
            pl.BlockSpec((RB, 1), lambda i: (i, 0)),
            pl.BlockSpec((1, hfull), lambda i: (0, 0)),
            pl.BlockSpec((hfull, hfull), lambda i: (0, 0)),
        ],
        out_specs=[
            pl.BlockSpec((RB, hfull), lambda i: (i, 0)),
            pl.BlockSpec((2, RB, hh), lambda i: (0, i, 0)),
        ],
        out_shape=[
            SDS((np_rows, hfull), F32),
            SDS((2, np_rows, hh), F32),
        ],
    )(s2, deg, b, w)


def _block_lin(x1, s2, deg, b2, wl, bl):
    np_rows, hh = s2.shape[1], s2.shape[2]
    hfull = 2 * hh
    grid = np_rows // RB

    def body(x1_ref, s_ref, d_ref, b2_ref, wl_ref, bl_ref, h_ref):
        dinv = lax.rsqrt(d_ref[...])
        x20 = _relu(s_ref[0] * dinv + b2_ref[0:1, :hh])
        x21 = _relu(s_ref[1] * dinv + b2_ref[0:1, hh:])
        x2 = jnp.concatenate([x20, x21], axis=1)
        r = jnp.dot(x1_ref[...], wl_ref[:hfull, :],
                    preferred_element_type=F32)
        r = r + jnp.dot(x2, wl_ref[hfull:, :], preferred_element_type=F32)
        h_ref[...] = _relu(r + bl_ref[...])

    return pl.pallas_call(
        body,
        grid=(grid,),
        in_specs=[
            pl.BlockSpec((RB, hfull), lambda i: (i, 0)),
            pl.BlockSpec((2, RB, hh), lambda i: (0, i, 0)),
            pl.BlockSpec((RB, 1), lambda i: (i, 0)),
            pl.BlockSpec((1, hfull), lambda i: (0, 0)),
            pl.BlockSpec((2 * hfull, hfull), lambda i: (0, 0)),
            pl.BlockSpec((1, hfull), lambda i: (0, 0)),
        ],
        out_specs=pl.BlockSpec((RB, hfull), lambda i: (i, 0)),
        out_shape=SDS((np_rows, hfull), F32),
    )(x1, s2, deg, b2, wl, bl)


def _sort_pool_select(keyb, batchb, num_graphs, k_top, k_pad):
    np_cols = keyb.shape[1]

    def body(key_ref, b_ref, idx_ref, val_ref):
        g = lax.broadcasted_iota(I32, (num_graphs, np_cols), 0)
        col = lax.broadcasted_iota(I32, (num_graphs, np_cols), 1)
        m = jnp.where(b_ref[...] == g, key_ref[...], NEG)
        idx_cols = []
        val_cols = []
        for _ in range(k_top):
            best = jnp.max(m, axis=1, keepdims=True)
            ok = best > (0.1 * NEG)
            sel = (m == best) & ok
            bidx = jnp.min(jnp.where(sel, col, np_cols), axis=1, keepdims=True)
            bidx = jnp.where(ok, bidx, 0)
            idx_cols.append(bidx)
            val_cols.append(ok.astype(F32))
            m = jnp.where(col == bidx, NEG, m)
        idx_cols.append(jnp.zeros((num_graphs, k_pad - k_top), I32))
        val_cols.append(jnp.zeros((num_graphs, k_pad - k_top), F32))
        idx_ref[...] = jnp.concatenate(idx_cols, axis=1)
        val_ref[...] = jnp.concatenate(val_cols, axis=1)

    return pl.pallas_call(
        body,
        out_shape=[
            SDS((num_graphs, k_pad), I32),
            SDS((num_graphs, k_pad), F32),
        ],
    )(keyb, batchb)


def _head(g3, v3, l1w, l1b, l2w, l2b, k_top):
    kp, ng, hfull = g3.shape
    nc = l2w.shape[1]

    def body(g_ref, v_ref, w1_ref, b1_ref, w2_ref, b2_ref, o_ref):
        acc = jnp.zeros((ng, hfull), F32)
        for r in range(k_top):
            pr = g_ref[r] * v_ref[r]
            acc = acc + jnp.dot(pr, w1_ref[pl.ds(r * hfull, hfull), :],
                                preferred_element_type=F32)
        p = _relu(acc + b1_ref[...])
        z = jnp.dot(p, w2_ref[...], preferred_element_type=F32) + b2_ref[...]
        zmax = jnp.max(z, axis=1, keepdims=True)
        e = jnp.exp(z - zmax)
        lse = jnp.log(jnp.sum(e, axis=1, keepdims=True))
        o_ref[...] = z - zmax - lse

    return pl.pallas_call(
        body,
        out_shape=SDS((ng, nc), F32),
    )(g3, v3, l1w, l1b, l2w, l2b)


def kernel(x, edge_index, batch,
           B0_W1, B0_b1, B0_W2, B0_b2, B0_Wl, B0_bl,
           B1_W1, B1_b1, B1_W2, B1_b2, B1_Wl, B1_bl,
           B2_W1, B2_b1, B2_W2, B2_b2, B2_Wl, B2_bl,
           lin1_W, lin1_b, lin2_W, lin2_b):
    n, fin = x.shape
    e = edge_index.shape[1]
    hfull = B0_W1.shape[1]
    hh = hfull // 2
    ng = 64
    k_top = lin1_W.shape[0] // hfull
    k_pad = 16
    nw = NSC * NSS

    np_rows = ((n + RB - 1) // RB) * RB
    align = nw * 8
    chunks = ((e + C - 1) // C + align - 1) // align * align
    ep = chunks * C

    xp = jnp.pad(x, ((0, np_rows - n), (0, 0)))
    n_pad_rows = np_rows - n
    pad_idx = (n + (jnp.arange(ep - e, dtype=I32) % n_pad_rows))
    src = jnp.concatenate([edge_index[0], pad_idx]).reshape(chunks, C)
    dst = jnp.concatenate([edge_index[1], pad_idx]).reshape(chunks, C)
    batchb = jnp.pad(batch, (0, np_rows - n),
                     constant_values=-1).reshape(1, np_rows)

    deg2 = _sc_degree(dst, np_rows)
    deg = (deg2[0] + deg2[1])[:, None]

    h = xp
    for (w1, b1, w2, b2, wl, bl) in (
            (B0_W1, B0_b1, B0_W2, B0_b2, B0_Wl, B0_bl),
            (B1_W1, B1_b1, B1_W2, B1_b2, B1_Wl, B1_bl),
            (B2_W1, B2_b1, B2_W2, B2_b2, B2_Wl, B2_bl)):
        u1 = _mm_scale(h, deg, w1)
        s1 = _sc_conv(u1, src, dst)
        x1, u2 = _act_mm(s1, deg, b1.reshape(1, hfull), w2)
        s2 = _sc_conv(u2, src, dst)
        h = _block_lin(x1, s2, deg, b2.reshape(1, hfull),
                       wl, bl.reshape(1, hfull))

    keyb = h[:, hfull - 1].reshape(1, np_rows)
    idx, valid = _sort_pool_select(keyb, batchb, ng, k_top, k_pad)

    idx_flat = idx.T.reshape(-1)
    rows = _sc_pool_gather(h, idx_flat)
    g3 = rows.reshape(k_pad, ng, hfull)
    v3 = valid.T.reshape(k_pad, ng, 1)

    return _head(g3, v3, lin1_W, lin1_b.reshape(1, hfull),
                 lin2_W, lin2_b.reshape(1, lin2_W.shape[1]), k_top)
